# Initial kernel scaffold; baseline (speedup 1.0000x reference)
import jax
import jax.numpy as jnp
from jax.experimental import pallas as pl


def kernel(context_ids, doc_ids, target_noise_ids, Doc, Word, Output):
    raise NotImplementedError("write your pallas kernel here")


# speedup vs baseline: 2.1459x; 2.1459x over previous
import functools

import jax
import jax.numpy as jnp
from jax import lax
from jax.experimental import pallas as pl
from jax.experimental.pallas import tpu as pltpu
from jax.experimental.pallas import tpu_sc as plsc

VEC_DIM = 64
BATCH = 16384
CTX = 20
N_NOISE = 10

NC = 2
NS = 16
NW = NC * NS
BPW = BATCH // NW
BPC = BATCH // NC

IDXW = 128
CTX_ROWS = BATCH * CTX // IDXW
TN_ROWS = BATCH * N_NOISE // IDXW
DOC_ROWS = BATCH // IDXW
CTX_RPT = CTX_ROWS // NW
TN_RPT = TN_ROWS // NW
DOC_RPT = DOC_ROWS // NW
RPC = 8
N_CTX_CHUNKS = CTX_RPT // RPC
N_TN_CHUNKS = TN_RPT // RPC

_mesh = plsc.VectorSubcoreMesh(core_axis_name="c", subcore_axis_name="s")


@functools.partial(
    pl.kernel,
    out_type=[
        jax.ShapeDtypeStruct((BATCH, VEC_DIM), jnp.float32),
        jax.ShapeDtypeStruct((BATCH * N_NOISE, VEC_DIM), jnp.float32),
    ],
    mesh=_mesh,
    scratch_types=[
        pltpu.VMEM_SHARED((BPC, VEC_DIM), jnp.float32),
        pltpu.VMEM((RPC * IDXW, VEC_DIM), jnp.float32),
        pltpu.VMEM((RPC, IDXW), jnp.int32),
        pltpu.VMEM((RPC, IDXW), jnp.int32),
        pltpu.VMEM((RPC, IDXW), jnp.int32),
        pltpu.SemaphoreType.DMA,
    ],
    compiler_params=pltpu.CompilerParams(use_tc_tiling_on_sc=False),
)
def _sc_gather(ctx_h, doc_h, tn_h, slot_h, doc_tbl, word_tbl, ot_tbl,
               x_h, or_h, xs, rows, cidx, slotv, didx, sem):
    sid = lax.axis_index("s")
    wid = lax.axis_index("c") * NS + sid

    pltpu.sync_copy(doc_h.at[pl.ds((wid // 2) * RPC, RPC)], didx)
    for k in range(DOC_RPT):
        pltpu.async_copy(
            doc_tbl.at[didx.at[(wid % 2) * DOC_RPT + k]],
            rows.at[pl.ds(k * IDXW, IDXW)],
            sem,
        ).wait()
    pltpu.sync_copy(rows.at[pl.ds(0, BPW)], xs.at[pl.ds(sid * BPW, BPW)])

    @pl.loop(0, N_CTX_CHUNKS)
    def _(c):
        rowbase = wid * CTX_RPT + c * RPC
        pltpu.sync_copy(ctx_h.at[pl.ds(rowbase, RPC)], cidx)
        pltpu.sync_copy(slot_h.at[pl.ds(rowbase, RPC)], slotv)
        for k in range(RPC):
            pltpu.async_copy(
                word_tbl.at[cidx.at[k]], rows.at[pl.ds(k * IDXW, IDXW)], sem
            ).wait()
        for k in range(RPC):
            pltpu.sync_copy(
                rows.at[pl.ds(k * IDXW, IDXW)], xs.at[slotv.at[k]], add=True
            )

    pltpu.sync_copy(xs.at[pl.ds(sid * BPW, BPW)], x_h.at[pl.ds(wid * BPW, BPW)])

    @pl.loop(0, N_TN_CHUNKS)
    def _(c):
        rowbase = wid * TN_RPT + c * RPC
        pltpu.sync_copy(tn_h.at[pl.ds(rowbase, RPC)], cidx)
        for k in range(RPC):
            pltpu.async_copy(
                ot_tbl.at[cidx.at[k]], rows.at[pl.ds(k * IDXW, IDXW)], sem
            ).wait()
        pltpu.sync_copy(rows, or_h.at[pl.ds(rowbase * IDXW, RPC * IDXW)])


_BB = 512


def _dot_body(x_ref, o_ref, out_ref):
    x = x_ref[...]
    o = o_ref[...]
    out_ref[...] = jnp.sum(o * x[:, None, :], axis=-1)


def _tc_dot(x, orows3):
    return pl.pallas_call(
        _dot_body,
        grid=(BATCH // _BB,),
        in_specs=[
            pl.BlockSpec((_BB, VEC_DIM), lambda i: (i, 0)),
            pl.BlockSpec((_BB, N_NOISE, VEC_DIM), lambda i: (i, 0, 0)),
        ],
        out_specs=pl.BlockSpec((_BB, N_NOISE), lambda i: (i, 0)),
        out_shape=jax.ShapeDtypeStruct((BATCH, N_NOISE), jnp.float32),
    )(x, orows3)


def kernel(context_ids, doc_ids, target_noise_ids, Doc, Word, Output):
    ctx2d = context_ids.astype(jnp.int32).reshape(CTX_ROWS, IDXW)
    doc2d = doc_ids.astype(jnp.int32).reshape(DOC_ROWS, IDXW)
    tn2d = target_noise_ids.astype(jnp.int32).reshape(TN_ROWS, IDXW)
    slot2d = (
        (jnp.arange(BATCH * CTX, dtype=jnp.int32) // CTX) % BPC
    ).reshape(CTX_ROWS, IDXW)
    out_t = Output.T

    x, orows = _sc_gather(ctx2d, doc2d, tn2d, slot2d, Doc, Word, out_t)
    return _tc_dot(x, orows.reshape(BATCH, N_NOISE, VEC_DIM))


# speedup vs baseline: 2.3977x; 1.1173x over previous
import functools

import jax
import jax.numpy as jnp
from jax import lax
from jax.experimental import pallas as pl
from jax.experimental.pallas import tpu as pltpu
from jax.experimental.pallas import tpu_sc as plsc

VEC_DIM = 64
BATCH = 16384
CTX = 20
N_NOISE = 10
NUM_WORDS1 = 100001

NC = 2
NS = 16
NW = NC * NS
BPW = BATCH // NW
BPC = BATCH // NC

IDXW = 128
SUB = 512
GPS = SUB // IDXW
CHUNK = 2 * SUB

CTX_IDS_PT = BPW * CTX
TN_IDS_PT = BPW * N_NOISE
N_CTX_CHUNKS = CTX_IDS_PT // CHUNK
N_TN_CHUNKS = TN_IDS_PT // CHUNK
SLOT_ROWS = BATCH * CTX // IDXW
SLOT_RPC = CHUNK // IDXW

_mesh = plsc.VectorSubcoreMesh(core_axis_name="c", subcore_axis_name="s")


@functools.partial(
    pl.kernel,
    out_type=[
        jax.ShapeDtypeStruct((BATCH, VEC_DIM), jnp.float32),
        jax.ShapeDtypeStruct((BATCH * N_NOISE, VEC_DIM), jnp.float32),
    ],
    mesh=_mesh,
    scratch_types=[
        pltpu.VMEM_SHARED((BPC, VEC_DIM), jnp.float32),
        pltpu.VMEM((SUB, VEC_DIM), jnp.float32),
        pltpu.VMEM((SUB, VEC_DIM), jnp.float32),
        pltpu.VMEM((CHUNK,), jnp.int32),
        pltpu.VMEM((SLOT_RPC, IDXW), jnp.int32),
        pltpu.SemaphoreType.DMA,
        pltpu.SemaphoreType.DMA,
        pltpu.SemaphoreType.DMA,
    ],
    compiler_params=pltpu.CompilerParams(use_tc_tiling_on_sc=False),
)
def _sc_gather(ctx_h, doc_h, tn_h, slot_h, doc_tbl, word_tbl, ot_tbl,
               x_h, or_h, xs, rows0, rows1, cidx, slotv, gsem0, gsem1, ssem):
    sid = lax.axis_index("s")
    wid = lax.axis_index("c") * NS + sid

    pltpu.sync_copy(doc_h.at[pl.ds(wid * BPW, BPW)], cidx.at[pl.ds(0, BPW)])
    gd = [
        pltpu.async_copy(
            doc_tbl.at[cidx.at[pl.ds(k * IDXW, IDXW)]],
            rows0.at[pl.ds(k * IDXW, IDXW)],
            gsem0,
        )
        for k in range(GPS)
    ]
    for g in gd:
        g.wait()
    pltpu.sync_copy(rows0, xs.at[pl.ds(sid * BPW, BPW)])

    @pl.loop(0, N_CTX_CHUNKS)
    def _(c):
        base = wid * CTX_IDS_PT + c * CHUNK
        pltpu.sync_copy(ctx_h.at[pl.ds(base, CHUNK)], cidx)
        pltpu.sync_copy(
            slot_h.at[pl.ds(wid * (CTX_IDS_PT // IDXW) + c * SLOT_RPC,
                            SLOT_RPC)],
            slotv,
        )
        g0 = [
            pltpu.async_copy(
                word_tbl.at[cidx.at[pl.ds(k * IDXW, IDXW)]],
                rows0.at[pl.ds(k * IDXW, IDXW)],
                gsem0,
            )
            for k in range(GPS)
        ]
        g1 = [
            pltpu.async_copy(
                word_tbl.at[cidx.at[pl.ds(SUB + k * IDXW, IDXW)]],
                rows1.at[pl.ds(k * IDXW, IDXW)],
                gsem1,
            )
            for k in range(GPS)
        ]
        for g in g0:
            g.wait()
        s0 = [
            pltpu.async_copy(
                rows0.at[pl.ds(k * IDXW, IDXW)],
                xs.at[slotv.at[k]],
                ssem,
                add=True,
            )
            for k in range(GPS)
        ]
        for g in g1:
            g.wait()
        s1 = [
            pltpu.async_copy(
                rows1.at[pl.ds(k * IDXW, IDXW)],
                xs.at[slotv.at[GPS + k]],
                ssem,
                add=True,
            )
            for k in range(GPS)
        ]
        for s in s0 + s1:
            s.wait()

    pltpu.sync_copy(xs.at[pl.ds(sid * BPW, BPW)], x_h.at[pl.ds(wid * BPW, BPW)])

    @pl.loop(0, N_TN_CHUNKS)
    def _(c):
        base = wid * TN_IDS_PT + c * CHUNK
        pltpu.sync_copy(tn_h.at[pl.ds(base, CHUNK)], cidx)
        g0 = [
            pltpu.async_copy(
                ot_tbl.at[cidx.at[pl.ds(k * IDXW, IDXW)]],
                rows0.at[pl.ds(k * IDXW, IDXW)],
                gsem0,
            )
            for k in range(GPS)
        ]
        g1 = [
            pltpu.async_copy(
                ot_tbl.at[cidx.at[pl.ds(SUB + k * IDXW, IDXW)]],
                rows1.at[pl.ds(k * IDXW, IDXW)],
                gsem1,
            )
            for k in range(GPS)
        ]
        for g in g0:
            g.wait()
        w0 = pltpu.async_copy(rows0, or_h.at[pl.ds(base, SUB)], ssem)
        for g in g1:
            g.wait()
        w1 = pltpu.async_copy(rows1, or_h.at[pl.ds(base + SUB, SUB)], ssem)
        w0.wait()
        w1.wait()


_TCOL = 512
_WPAD = 196 * _TCOL


def _tr_body(o_ref, ot_ref):
    ot_ref[...] = o_ref[...].T


def _tc_transpose(output):
    return pl.pallas_call(
        _tr_body,
        grid=(_WPAD // _TCOL,),
        in_specs=[pl.BlockSpec((VEC_DIM, _TCOL), lambda i: (0, i))],
        out_specs=pl.BlockSpec((_TCOL, VEC_DIM), lambda i: (i, 0)),
        out_shape=jax.ShapeDtypeStruct((_WPAD, VEC_DIM), jnp.float32),
    )(output)


_BB = 512


def _dot_body(x_ref, o_ref, out_ref):
    x = x_ref[...]
    o = o_ref[...]
    out_ref[...] = jnp.sum(o * x[:, None, :], axis=-1)


def _tc_dot(x, orows3):
    return pl.pallas_call(
        _dot_body,
        grid=(BATCH // _BB,),
        in_specs=[
            pl.BlockSpec((_BB, VEC_DIM), lambda i: (i, 0)),
            pl.BlockSpec((_BB, N_NOISE, VEC_DIM), lambda i: (i, 0, 0)),
        ],
        out_specs=pl.BlockSpec((_BB, N_NOISE), lambda i: (i, 0)),
        out_shape=jax.ShapeDtypeStruct((BATCH, N_NOISE), jnp.float32),
    )(x, orows3)


def kernel(context_ids, doc_ids, target_noise_ids, Doc, Word, Output):
    ctx_flat = context_ids.astype(jnp.int32).reshape(-1)
    tn_flat = target_noise_ids.astype(jnp.int32).reshape(-1)
    doc_flat = doc_ids.astype(jnp.int32)
    slot2d = (
        (jnp.arange(BATCH * CTX, dtype=jnp.int32) // CTX) % BPC
    ).reshape(SLOT_ROWS, IDXW)
    out_t = _tc_transpose(Output)

    x, orows = _sc_gather(ctx_flat, doc_flat, tn_flat, slot2d,
                          Doc, Word, out_t)
    return _tc_dot(x, orows.reshape(BATCH, N_NOISE, VEC_DIM))
